# baseline (speedup 1.0000x reference)
import functools

import jax
import jax.numpy as jnp
from jax import lax
from jax.experimental import pallas as pl
from jax.experimental.pallas import tpu as pltpu
from jax.experimental.pallas import tpu_sc as plsc

_N = 16384
_D = 4096
_L = 16
_NC = 2
_NS = 16
_NW = _NC * _NS
_RPT = _N // _NW
_RB = 8
_NBLK = _RPT // _RB
_G = 8
_NG = _D // (_G * _L)

_mesh = plsc.VectorSubcoreMesh(core_axis_name="c", subcore_axis_name="s")

_GDN = lax.GatherDimensionNumbers(
    offset_dims=(), collapsed_slice_dims=(0,), start_index_map=(0,))


def _bcast_lane(v, r):
    idx = jnp.full((_L, 1), r, jnp.int32)
    return lax.gather(v, idx, _GDN, slice_sizes=(1,),
                      mode=lax.GatherScatterMode.PROMISE_IN_BOUNDS)


@functools.partial(
    pl.kernel,
    mesh=_mesh,
    out_type=jax.ShapeDtypeStruct((_NW, 3 * _D), jnp.float32),
    scratch_types=[
        pltpu.VMEM((_RB, _D), jnp.float32),
        pltpu.VMEM((3 * _D,), jnp.float32),
        pltpu.VMEM((_RPT,), jnp.float32),
        pltpu.VMEM((_RPT,), jnp.float32),
        pltpu.VMEM((_RPT + _L,), jnp.float32),
        pltpu.VMEM((_RPT + _L,), jnp.float32),
    ],
)
def _sc_partial_sums(p_hbm, y_hbm, yp_hbm, out_hbm, buf, acc, ybuf, ypbuf, w1, w2):
    wid = lax.axis_index("s") * _NC + lax.axis_index("c")
    base = wid * _RPT

    pltpu.sync_copy(y_hbm.at[pl.ds(base, _RPT)], ybuf)
    pltpu.sync_copy(yp_hbm.at[pl.ds(base, _RPT)], ypbuf)

    zeros16 = jnp.zeros((_L,), jnp.float32)
    ones16 = jnp.ones((_L,), jnp.float32)

    def _wbody(i, _):
        o = i * _L
        w1[pl.ds(o, _L)] = jnp.where(ybuf[pl.ds(o, _L)] >= 0.0, ones16, zeros16)
        w2[pl.ds(o, _L)] = jnp.where(ypbuf[pl.ds(o, _L)] >= 0.0, ones16, zeros16)
        return _
    lax.fori_loop(0, _RPT // _L, _wbody, None)
    w1[pl.ds(_RPT, _L)] = zeros16
    w2[pl.ds(_RPT, _L)] = zeros16

    def _zbody(i, _):
        acc[pl.ds(i * _L, _L)] = zeros16
        return _
    lax.fori_loop(0, 3 * _D // _L, _zbody, None)

    def _blk_body(blk, _):
        pltpu.sync_copy(p_hbm.at[pl.ds(base + blk * _RB, _RB)], buf)
        w1v = w1[pl.ds(blk * _RB, _L)]
        w2v = w2[pl.ds(blk * _RB, _L)]

        def _g_body(g, _g):
            col0 = g * (_G * _L)
            a_all = [acc[pl.ds(col0 + k * _L, _L)] for k in range(_G)]
            a_pos = [acc[pl.ds(_D + col0 + k * _L, _L)] for k in range(_G)]
            a_pp = [acc[pl.ds(2 * _D + col0 + k * _L, _L)] for k in range(_G)]
            for r in range(_RB):
                b1 = _bcast_lane(w1v, r)
                b2 = _bcast_lane(w2v, r)
                for k in range(_G):
                    v = buf[r, pl.ds(col0 + k * _L, _L)]
                    a_all[k] = a_all[k] + v
                    a_pos[k] = a_pos[k] + v * b1
                    a_pp[k] = a_pp[k] + v * b2
            for k in range(_G):
                acc[pl.ds(col0 + k * _L, _L)] = a_all[k]
                acc[pl.ds(_D + col0 + k * _L, _L)] = a_pos[k]
                acc[pl.ds(2 * _D + col0 + k * _L, _L)] = a_pp[k]
            return _g
        lax.fori_loop(0, _NG, _g_body, None)
        return _
    lax.fori_loop(0, _NBLK, _blk_body, None)

    pltpu.sync_copy(acc, out_hbm.at[wid])


def _finish_body(part_ref, y_ref, yp_ref, out_ref):
    red = jnp.sum(part_ref[...], axis=0)
    s_all = red[0:_D]
    s_pos = red[_D:2 * _D]
    s_pp = red[2 * _D:3 * _D]
    y = y_ref[...]
    yp = yp_ref[...]
    n = jnp.float32(_N)
    n_pos = jnp.sum((y >= 0.0).astype(jnp.float32))
    n_pp = jnp.sum((yp >= 0.0).astype(jnp.float32))
    n_neg = n - n_pos

    pos_avg = s_pos / n_pos
    neg_avg = (s_all - s_pos) / n_neg
    pos_avg_p = s_pp / n_pp
    neg_avg_p = (s_all - s_pp) / (n - n_pp)

    def one_minus_cos(a, b):
        dot = jnp.sum(a * b)
        na = jnp.sqrt(jnp.sum(a * a))
        nb = jnp.sqrt(jnp.sum(b * b))
        return 1.0 - dot / jnp.maximum(na * nb, 1e-8)

    cp = one_minus_cos(pos_avg, pos_avg_p)
    cn = one_minus_cos(neg_avg, neg_avg_p)
    out_ref[0] = n_pos * cp / n + n_neg * cn / n


@jax.jit
def kernel(p_v, y, y_pred):
    partial = _sc_partial_sums(p_v, y, y_pred)
    out = pl.pallas_call(
        _finish_body,
        out_specs=pl.BlockSpec(memory_space=pltpu.SMEM),
        out_shape=jax.ShapeDtypeStruct((1,), jnp.float32),
    )(partial, y, y_pred)
    return out
